# baseline (speedup 1.0000x reference)
import functools

import jax
import jax.numpy as jnp
from jax import lax
from jax.experimental import pallas as pl
from jax.experimental.pallas import tpu as pltpu
from jax.experimental.pallas import tpu_sc as plsc

NC, NS = 2, 16
NW = NC * NS
NB = 8
PD = 128
PH = 56


@functools.lru_cache(maxsize=None)
def _build(BATCH, HIST, D):
    b_per_w = BATCH // NW
    n_chunks = b_per_w // NB

    mesh = plsc.VectorSubcoreMesh(
        core_axis_name="c", subcore_axis_name="s",
        num_cores=NC, num_subcores=NS)

    @functools.partial(
        pl.kernel,
        mesh=mesh,
        compiler_params=pltpu.CompilerParams(use_tc_tiling_on_sc=True),
        out_type=jax.ShapeDtypeStruct((BATCH, HIST, D), jnp.float32),
        scratch_types=[
            pltpu.VMEM((NB, PH), jnp.int32),
            pltpu.VMEM((NB, PH, PD), jnp.float32),
            pltpu.VMEM((NB, HIST, D), jnp.float32),
            pltpu.SemaphoreType.DMA,
            pltpu.SemaphoreType.DMA,
        ],
    )
    def gather_kernel(idx_hbm, table_hbm, out_hbm, idx_v, rows_v, comp_v,
                      gsem, osem):
        wid = lax.axis_index("s") * NC + lax.axis_index("c")
        b_base = wid * b_per_w

        def body(c, carry):
            b0 = b_base + c * NB
            pltpu.sync_copy(idx_hbm.at[pl.ds(b0, NB)], idx_v)
            for i in range(NB):
                pltpu.async_copy(
                    table_hbm.at[idx_v.at[i]], rows_v.at[i], gsem)
            for i in range(NB):
                pltpu.make_async_copy(
                    table_hbm.at[pl.ds(0, PH)], rows_v.at[i], gsem).wait()

            def comp_row(h, carry2):
                for i in range(NB):
                    for j in range(D // 16):
                        comp_v[i, h, pl.ds(j * 16, 16)] = (
                            rows_v[i, h, pl.ds(j * 16, 16)])
                return carry2

            lax.fori_loop(0, HIST, comp_row, 0)

            pltpu.async_copy(comp_v, out_hbm.at[pl.ds(b0, NB)], osem)
            pltpu.make_async_copy(
                out_hbm.at[pl.ds(b0, NB)], comp_v, osem).wait()
            return carry

        lax.fori_loop(0, n_chunks, body, 0)

    return gather_kernel


def kernel(token_ids, w):
    BATCH, HIST = token_ids.shape
    V, D = w.shape
    idx2 = jnp.pad(token_ids.astype(jnp.int32), ((0, 0), (0, PH - HIST)),
                   mode="edge")
    w_pad = jnp.pad(w, ((0, 0), (0, PD - D)))
    return _build(BATCH, HIST, D)(idx2, w_pad)
